# Initial kernel scaffold; baseline (speedup 1.0000x reference)
import jax
import jax.numpy as jnp
from jax.experimental import pallas as pl


def kernel(x, edge_index, batch, Wc, bc, gamma, beta, rm, rv, Wm1, bm1, Wm2, bm2, Wm3, bm3):
    raise NotImplementedError("write your pallas kernel here")


# speedup vs baseline: 6.4217x; 6.4217x over previous
import functools

import jax
import jax.numpy as jnp
from jax import lax
from jax.experimental import pallas as pl
from jax.experimental.pallas import tpu as pltpu
from jax.experimental.pallas import tpu_sc as plsc

F = 128
H = 128
G = 8
BM = 256
K = 128
N_TILES = 32
N_SUB = 16


def _pad_up(n, m):
    return (n + m - 1) // m * m


def _deg_kernel(npad, ep):
    pt = ep // N_TILES
    rows_pt = npad // N_SUB
    mesh = plsc.VectorSubcoreMesh(core_axis_name="c", subcore_axis_name="s")

    @functools.partial(
        pl.kernel,
        out_type=jax.ShapeDtypeStruct((2, npad, H), jnp.float32),
        mesh=mesh,
        scratch_types=[
            pltpu.VMEM((K,), jnp.int32),
            pltpu.VMEM((K, H), jnp.float32),
            pltpu.VMEM_SHARED((npad, H), jnp.float32),
            pltpu.SemaphoreType.DMA,
        ],
    )
    def deg(dst_hbm, ones_hbm, zeros_hbm, out_hbm, di, ones_v, acc, sem):
        c = lax.axis_index("c")
        s = lax.axis_index("s")
        tid = c * N_SUB + s

        pltpu.sync_copy(ones_hbm, ones_v)
        pltpu.sync_copy(zeros_hbm, acc.at[pl.ds(s * rows_pt, rows_pt)])
        plsc.subcore_barrier()
        t0 = tid * pt

        def chunk(i, carry):
            base = t0 + i * K
            pltpu.sync_copy(dst_hbm.at[pl.ds(base, K)], di)
            pltpu.sync_copy(ones_v, acc.at[di], add=True)
            return carry

        lax.fori_loop(0, pt // K, chunk, 0)
        plsc.subcore_barrier()
        pltpu.sync_copy(acc.at[pl.ds(s * rows_pt, rows_pt)],
                        out_hbm.at[c, pl.ds(s * rows_pt, rows_pt)])

    return deg


def _scatter_kernel(npad, ep):
    pt = ep // N_TILES
    rows_pt = npad // N_SUB
    mesh = plsc.VectorSubcoreMesh(core_axis_name="c", subcore_axis_name="s")

    @functools.partial(
        pl.kernel,
        out_type=jax.ShapeDtypeStruct((2, npad, H), jnp.float32),
        mesh=mesh,
        scratch_types=[
            pltpu.VMEM((K,), jnp.int32),
            pltpu.VMEM((K,), jnp.int32),
            pltpu.VMEM((K, H), jnp.float32),
            pltpu.VMEM_SHARED((npad, H), jnp.float32),
            pltpu.SemaphoreType.DMA,
        ],
    )
    def scat(y_hbm, src_hbm, dst_hbm, zeros_hbm, out_hbm, si, di, rows, acc,
             sem):
        c = lax.axis_index("c")
        s = lax.axis_index("s")
        tid = c * N_SUB + s

        pltpu.sync_copy(zeros_hbm, acc.at[pl.ds(s * rows_pt, rows_pt)])
        plsc.subcore_barrier()
        t0 = tid * pt

        def chunk(i, carry):
            base = t0 + i * K
            pltpu.sync_copy(src_hbm.at[pl.ds(base, K)], si)
            pltpu.sync_copy(dst_hbm.at[pl.ds(base, K)], di)
            pltpu.async_copy(y_hbm.at[si], rows, sem).wait()
            pltpu.sync_copy(rows, acc.at[di], add=True)
            return carry

        lax.fori_loop(0, pt // K, chunk, 0)
        plsc.subcore_barrier()
        pltpu.sync_copy(acc.at[pl.ds(s * rows_pt, rows_pt)],
                        out_hbm.at[c, pl.ds(s * rows_pt, rows_pt)])

    return scat


def _dinv_of(d0, d1):
    return lax.rsqrt(d0[:, 0:1] + d1[:, 0:1] + 1.0)


def _dot(a, b):
    return jnp.dot(a, b, preferred_element_type=jnp.float32,
                   precision=lax.Precision.HIGHEST)


def _t0_body(x_ref, w_ref, d0_ref, d1_ref, y_ref):
    dinv = _dinv_of(d0_ref[...], d1_ref[...])
    y_ref[...] = _dot(x_ref[...], w_ref[...]) * dinv


def _tmid_body(p0_ref, p1_ref, yp_ref, d0_ref, d1_ref, w_ref,
               rm_ref, rv_ref, ga_ref, be_ref, bc_ref, y_ref):
    dinv = _dinv_of(d0_ref[...], d1_ref[...])
    agg = (p0_ref[...] + p1_ref[...] + yp_ref[...]) * dinv + bc_ref[...]
    hb = (agg - rm_ref[...]) * lax.rsqrt(rv_ref[...] + 1e-5) * ga_ref[...] \
        + be_ref[...]
    h = jnp.maximum(hb, 0.0)
    y_ref[...] = _dot(h, w_ref[...]) * dinv


def _tfin_body(p0_ref, p1_ref, yp_ref, d0_ref, d1_ref, b_ref,
               rm_ref, rv_ref, ga_ref, be_ref, bc_ref,
               w1_ref, b1_ref, w2_ref, b2_ref, w3_ref, b3_ref,
               out_ref, pool_acc, cnt_acc):
    i = pl.program_id(0)
    nsteps = pl.num_programs(0)

    @pl.when(i == 0)
    def _init():
        pool_acc[...] = jnp.zeros_like(pool_acc)
        cnt_acc[...] = jnp.zeros_like(cnt_acc)

    dinv = _dinv_of(d0_ref[...], d1_ref[...])
    agg = (p0_ref[...] + p1_ref[...] + yp_ref[...]) * dinv + bc_ref[...]
    hb = (agg - rm_ref[...]) * lax.rsqrt(rv_ref[...] + 1e-5) * ga_ref[...] \
        + be_ref[...]
    h = jnp.maximum(hb, 0.0)

    gids = lax.broadcasted_iota(jnp.int32, (G, BM), 0)
    onehot = (b_ref[...] == gids).astype(jnp.float32)
    pool_acc[...] += _dot(onehot, h)
    cnt_acc[...] += jnp.broadcast_to(
        jnp.sum(onehot, axis=1, keepdims=True), (G, H))

    @pl.when(i == nsteps - 1)
    def _finish():
        pooled = pool_acc[...] / jnp.maximum(cnt_acc[...], 1.0)
        z = jnp.maximum(_dot(pooled, w1_ref[...]) + b1_ref[...], 0.0)
        z = jnp.maximum(_dot(z, w2_ref[...]) + b2_ref[...], 0.0)
        out_ref[...] = _dot(z, w3_ref[...]) + b3_ref[...]


def _row_spec(w):
    return pl.BlockSpec((BM, w), lambda i: (i, 0))


def _full_spec(r, c):
    return pl.BlockSpec((r, c), lambda i: (0, 0))


def kernel(x, edge_index, batch, Wc, bc, gamma, beta, rm, rv,
           Wm1, bm1, Wm2, bm2, Wm3, bm3):
    n, f = x.shape
    e = edge_index.shape[1]
    npad = _pad_up(n + 1, BM)
    ep = _pad_up(e, N_TILES * K)

    xp = jnp.zeros((npad, f), jnp.float32).at[:n].set(x)
    srcp = jnp.concatenate(
        [edge_index[0].astype(jnp.int32),
         jnp.zeros((ep - e,), jnp.int32)])
    dstp = jnp.concatenate(
        [edge_index[1].astype(jnp.int32),
         jnp.full((ep - e,), n, jnp.int32)])
    bpad = jnp.full((1, npad), G, jnp.int32).at[0, :n].set(
        batch.astype(jnp.int32))

    zeros_w = jnp.zeros((npad // N_SUB, H), jnp.float32)
    ones_w = jnp.ones((K, H), jnp.float32)

    w2p = jnp.zeros((H, H), jnp.float32).at[:, :Wm2.shape[1]].set(Wm2)
    b2p = jnp.zeros((1, H), jnp.float32).at[0, :Wm2.shape[1]].set(bm2)
    w3p = jnp.zeros((H, H), jnp.float32).at[:Wm3.shape[0], 0].set(Wm3[:, 0])
    b3p = jnp.zeros((1, H), jnp.float32).at[0, 0].set(bm3[0])
    w1p = Wm1
    b1p = bm1.reshape(1, H)

    params = [(rm[i].reshape(1, H), rv[i].reshape(1, H),
               gamma[i].reshape(1, H), beta[i].reshape(1, H),
               bc[i].reshape(1, H)) for i in range(3)]

    grid = (npad // BM,)

    degp = _deg_kernel(npad, ep)(dstp, ones_w, zeros_w)
    d0, d1 = degp[0], degp[1]

    y = pl.pallas_call(
        _t0_body,
        grid=grid,
        in_specs=[_row_spec(f), _full_spec(f, H), _row_spec(H), _row_spec(H)],
        out_specs=_row_spec(H),
        out_shape=jax.ShapeDtypeStruct((npad, H), jnp.float32),
    )(xp, Wc[0], d0, d1)

    for i in range(2):
        part = _scatter_kernel(npad, ep)(y, srcp, dstp, zeros_w)
        rm_i, rv_i, ga_i, be_i, bc_i = params[i]
        y = pl.pallas_call(
            _tmid_body,
            grid=grid,
            in_specs=[_row_spec(H), _row_spec(H), _row_spec(H),
                      _row_spec(H), _row_spec(H), _full_spec(H, H),
                      _full_spec(1, H), _full_spec(1, H), _full_spec(1, H),
                      _full_spec(1, H), _full_spec(1, H)],
            out_specs=_row_spec(H),
            out_shape=jax.ShapeDtypeStruct((npad, H), jnp.float32),
        )(part[0], part[1], y, d0, d1, Wc[i + 1],
          rm_i, rv_i, ga_i, be_i, bc_i)

    part = _scatter_kernel(npad, ep)(y, srcp, dstp, zeros_w)
    rm_i, rv_i, ga_i, be_i, bc_i = params[2]
    out = pl.pallas_call(
        _tfin_body,
        grid=grid,
        in_specs=[_row_spec(H), _row_spec(H), _row_spec(H),
                  _row_spec(H), _row_spec(H),
                  pl.BlockSpec((1, BM), lambda i: (0, i)),
                  _full_spec(1, H), _full_spec(1, H), _full_spec(1, H),
                  _full_spec(1, H), _full_spec(1, H),
                  _full_spec(H, H), _full_spec(1, H), _full_spec(H, H),
                  _full_spec(1, H), _full_spec(H, H), _full_spec(1, H)],
        out_specs=_full_spec(G, H),
        out_shape=jax.ShapeDtypeStruct((G, H), jnp.float32),
        scratch_shapes=[pltpu.VMEM((G, H), jnp.float32),
                        pltpu.VMEM((G, H), jnp.float32)],
    )(part[0], part[1], y, d0, d1, bpad,
      rm_i, rv_i, ga_i, be_i, bc_i,
      w1p, b1p, w2p, b2p, w3p, b3p)

    return out[:, 0]
